# Initial kernel scaffold; baseline (speedup 1.0000x reference)
import jax
import jax.numpy as jnp
from jax.experimental import pallas as pl


def kernel(inputs):
    raise NotImplementedError("write your pallas kernel here")


# speedup vs baseline: 1.2763x; 1.2763x over previous
import functools

import jax
import jax.numpy as jnp
from jax import lax
from jax.experimental import pallas as pl
from jax.experimental.pallas import tpu as pltpu
from jax.experimental.pallas import tpu_sc as plsc

L = 16
NC, NS = 2, 16
NW = NC * NS
TOTAL = 128 * 32768
PER_W = TOTAL // NW
CHUNK = 16384
NCHUNK = PER_W // CHUNK


def _shuffle(x, idx):
    return lax.gather(
        x,
        idx[:, None],
        lax.GatherDimensionNumbers(
            offset_dims=(), collapsed_slice_dims=(0,), start_index_map=(0,)
        ),
        slice_sizes=(1,),
        mode=lax.GatherScatterMode.PROMISE_IN_BOUNDS,
    )


def _lwta_vreg(x):
    iota = lax.iota(jnp.int32, L)
    i1 = iota ^ 1
    i2 = iota ^ 2
    pos = iota & 3
    m = jnp.maximum(x, _shuffle(x, i1))
    m = jnp.maximum(m, _shuffle(m, i2))
    score = jnp.where(x == m, pos, L)
    sm = jnp.minimum(score, _shuffle(score, i1))
    sm = jnp.minimum(sm, _shuffle(sm, i2))
    return jnp.where(score == sm, x, 0.0)


@functools.partial(
    pl.kernel,
    mesh=plsc.VectorSubcoreMesh(core_axis_name="c", subcore_axis_name="s"),
    out_type=jax.ShapeDtypeStruct((TOTAL,), jnp.float32),
    scratch_types=[
        pltpu.VMEM((CHUNK,), jnp.float32),
        pltpu.VMEM((CHUNK,), jnp.float32),
    ],
)
def _lwta_sc(x_hbm, o_hbm, in_v, out_v):
    wid = lax.axis_index("s") * NC + lax.axis_index("c")
    base = wid * PER_W

    def chunk_body(c, _):
        off = base + c * CHUNK
        pltpu.sync_copy(x_hbm.at[pl.ds(off, CHUNK)], in_v)

        def vreg_body(j, _):
            x = in_v[pl.ds(j * L, L)]
            out_v[pl.ds(j * L, L)] = _lwta_vreg(x)
            return 0

        lax.fori_loop(0, CHUNK // L, vreg_body, 0)
        pltpu.sync_copy(out_v, o_hbm.at[pl.ds(off, CHUNK)])
        return 0

    lax.fori_loop(0, NCHUNK, chunk_body, 0)


def kernel(inputs):
    flat = inputs.reshape(TOTAL)
    out = _lwta_sc(flat)
    return out.reshape(inputs.shape)


# speedup vs baseline: 1.5299x; 1.1987x over previous
import functools

import jax
import jax.numpy as jnp
from jax import lax
from jax.experimental import pallas as pl
from jax.experimental.pallas import tpu as pltpu
from jax.experimental.pallas import tpu_sc as plsc

L = 16
NC, NS = 2, 16
NW = NC * NS
TOTAL = 128 * 32768
PER_W = TOTAL // NW
CHUNK = 16384
NCHUNK = PER_W // CHUNK


def _shuffle(x, idx):
    return lax.gather(
        x,
        idx[:, None],
        lax.GatherDimensionNumbers(
            offset_dims=(), collapsed_slice_dims=(0,), start_index_map=(0,)
        ),
        slice_sizes=(1,),
        mode=lax.GatherScatterMode.PROMISE_IN_BOUNDS,
    )


def _lwta_vreg(x):
    iota = lax.iota(jnp.int32, L)
    i1 = iota ^ 1
    i2 = iota ^ 2
    pos = iota & 3
    m = jnp.maximum(x, _shuffle(x, i1))
    m = jnp.maximum(m, _shuffle(m, i2))
    score = jnp.where(x == m, pos, L)
    sm = jnp.minimum(score, _shuffle(score, i1))
    sm = jnp.minimum(sm, _shuffle(sm, i2))
    return jnp.where(score == sm, x, 0.0)


UNROLL = 8


def _compute_chunk(in_v, out_v):
    def body(j, _):
        o = j * (UNROLL * L)
        for k in range(UNROLL):
            s = pl.ds(o + k * L, L)
            out_v[s] = _lwta_vreg(in_v[s])
        return 0

    lax.fori_loop(0, CHUNK // (UNROLL * L), body, 0)


@functools.partial(
    pl.kernel,
    mesh=plsc.VectorSubcoreMesh(core_axis_name="c", subcore_axis_name="s"),
    out_type=jax.ShapeDtypeStruct((TOTAL,), jnp.float32),
    scratch_types=[
        pltpu.VMEM((CHUNK,), jnp.float32),
        pltpu.VMEM((CHUNK,), jnp.float32),
        pltpu.VMEM((CHUNK,), jnp.float32),
        pltpu.VMEM((CHUNK,), jnp.float32),
        pltpu.SemaphoreType.DMA,
        pltpu.SemaphoreType.DMA,
        pltpu.SemaphoreType.DMA,
        pltpu.SemaphoreType.DMA,
    ],
)
def _lwta_sc(x_hbm, o_hbm, in0, in1, out0, out1, s_in0, s_in1, s_out0, s_out1):
    wid = lax.axis_index("s") * NC + lax.axis_index("c")
    base = wid * PER_W
    ins, outs = [in0, in1], [out0, out1]
    s_ins, s_outs = [s_in0, s_in1], [s_out0, s_out1]
    in_h = [None] * NCHUNK
    out_h = [None] * NCHUNK

    in_h[0] = pltpu.async_copy(x_hbm.at[pl.ds(base, CHUNK)], ins[0], s_ins[0])
    for c in range(NCHUNK):
        b = c % 2
        if c + 1 < NCHUNK:
            nb = (c + 1) % 2
            in_h[c + 1] = pltpu.async_copy(
                x_hbm.at[pl.ds(base + (c + 1) * CHUNK, CHUNK)], ins[nb], s_ins[nb]
            )
        in_h[c].wait()
        if c >= 2:
            out_h[c - 2].wait()
        _compute_chunk(ins[b], outs[b])
        out_h[c] = pltpu.async_copy(
            outs[b], o_hbm.at[pl.ds(base + c * CHUNK, CHUNK)], s_outs[b]
        )
    out_h[NCHUNK - 2].wait()
    out_h[NCHUNK - 1].wait()


def kernel(inputs):
    flat = inputs.reshape(TOTAL)
    out = _lwta_sc(flat)
    return out.reshape(inputs.shape)


# speedup vs baseline: 2.7254x; 1.7815x over previous
import functools

import jax
import jax.numpy as jnp
from jax import lax
from jax.experimental import pallas as pl
from jax.experimental.pallas import tpu as pltpu
from jax.experimental.pallas import tpu_sc as plsc

L = 16
NC, NS = 2, 16
NW = NC * NS
TOTAL = 128 * 32768
PER_W = TOTAL // NW
CHUNK = 16384
NCHUNK = PER_W // CHUNK


def _shuffle(x, idx):
    return lax.gather(
        x,
        idx[:, None],
        lax.GatherDimensionNumbers(
            offset_dims=(), collapsed_slice_dims=(0,), start_index_map=(0,)
        ),
        slice_sizes=(1,),
        mode=lax.GatherScatterMode.PROMISE_IN_BOUNDS,
    )


def _lwta_vreg(x):
    iota = lax.iota(jnp.int32, L)
    i1 = iota ^ 1
    i2 = iota ^ 2
    pos = iota & 3
    m = jnp.maximum(x, _shuffle(x, i1))
    m = jnp.maximum(m, _shuffle(m, i2))
    score = jnp.where(x == m, pos, L)
    sm = jnp.minimum(score, _shuffle(score, i1))
    sm = jnp.minimum(sm, _shuffle(sm, i2))
    return jnp.where(score == sm, x, 0.0)


UNROLL = 8


def _compute_chunk(in_v, out_v):
    def body(j, _):
        o = j * (UNROLL * L)
        for k in range(UNROLL):
            s = pl.ds(o + k * L, L)
            out_v[s] = _lwta_vreg(in_v[s])
        return 0

    lax.fori_loop(0, CHUNK // (UNROLL * L), body, 0)


B, D = 128, 32768
ROWS_PER_W = B // NW
CHUNKS_PER_ROW = D // CHUNK


@functools.partial(
    pl.kernel,
    mesh=plsc.VectorSubcoreMesh(core_axis_name="c", subcore_axis_name="s"),
    out_type=jax.ShapeDtypeStruct((B, D), jnp.float32),
    scratch_types=[
        pltpu.VMEM((CHUNK,), jnp.float32),
        pltpu.VMEM((CHUNK,), jnp.float32),
        pltpu.VMEM((CHUNK,), jnp.float32),
        pltpu.VMEM((CHUNK,), jnp.float32),
        pltpu.SemaphoreType.DMA,
        pltpu.SemaphoreType.DMA,
        pltpu.SemaphoreType.DMA,
        pltpu.SemaphoreType.DMA,
    ],
)
def _lwta_sc(x_hbm, o_hbm, in0, in1, out0, out1, s_in0, s_in1, s_out0, s_out1):
    wid = lax.axis_index("s") * NC + lax.axis_index("c")
    row0 = wid * ROWS_PER_W
    ins, outs = [in0, in1], [out0, out1]
    s_ins, s_outs = [s_in0, s_in1], [s_out0, s_out1]
    in_h = [None] * NCHUNK
    out_h = [None] * NCHUNK

    def src(c):
        return x_hbm.at[row0 + c // CHUNKS_PER_ROW,
                        pl.ds((c % CHUNKS_PER_ROW) * CHUNK, CHUNK)]

    def dst(c):
        return o_hbm.at[row0 + c // CHUNKS_PER_ROW,
                        pl.ds((c % CHUNKS_PER_ROW) * CHUNK, CHUNK)]

    in_h[0] = pltpu.async_copy(src(0), ins[0], s_ins[0])
    for c in range(NCHUNK):
        b = c % 2
        if c + 1 < NCHUNK:
            nb = (c + 1) % 2
            in_h[c + 1] = pltpu.async_copy(src(c + 1), ins[nb], s_ins[nb])
        in_h[c].wait()
        if c >= 2:
            out_h[c - 2].wait()
        _compute_chunk(ins[b], outs[b])
        out_h[c] = pltpu.async_copy(outs[b], dst(c), s_outs[b])
    out_h[NCHUNK - 2].wait()
    out_h[NCHUNK - 1].wait()


def kernel(inputs):
    return _lwta_sc(inputs)
